# baseline (speedup 1.0000x reference)
import jax
import jax.numpy as jnp
from jax.experimental import pallas as pl
from jax.experimental.pallas import tpu as pltpu

_B, _C, _L, _D = 64, 32, 4, 2048
_PRED = 720
_PS = 8
_DBLK = 256
_NK = _D // _DBLK
_NB = 4
_BB = _B // _NB
_ROWS = _BB * _C


def _head_kernel(gates_ref, bmat_ref, *refs):
    xs = refs[:_PS]
    ws = refs[_PS:2 * _PS]
    out_ref = refs[2 * _PS]
    acc_ref = refs[2 * _PS + 1]
    k = pl.program_id(1)

    g = jnp.maximum(gates_ref[...], 0.0)
    acc = jnp.zeros((_ROWS, _PRED), jnp.float32)
    for i in range(_PS):
        x = xs[i][...].reshape(_BB, _C, _DBLK)
        gi = g[:, i].reshape(_BB, 1, 1)
        gx = (x * gi).reshape(_ROWS, _DBLK)
        acc = acc + jax.lax.dot_general(
            gx, ws[i][...],
            (((1,), (1,)), ((), ())),
            preferred_element_type=jnp.float32,
        )

    @pl.when(k == 0)
    def _():
        acc_ref[...] = acc

    @pl.when(k != 0)
    def _():
        acc_ref[...] = acc_ref[...] + acc

    @pl.when(k == _NK - 1)
    def _():
        bias = jax.lax.dot_general(
            g, bmat_ref[...],
            (((1,), (0,)), ((), ())),
            preferred_element_type=jnp.float32,
        )
        total = acc_ref[...].reshape(_BB, _C, _PRED) + bias[:, None, :] + 1e-9
        out_ref[...] = jnp.transpose(total, (0, 2, 1))


def kernel(xs_0, xs_1, xs_2, xs_3, xs_4, xs_5, xs_6, xs_7, gates,
           W_0, W_1, W_2, W_3, W_4, W_5, W_6, W_7,
           b_0, b_1, b_2, b_3, b_4, b_5, b_6, b_7):
    xs = [xs_0, xs_1, xs_2, xs_3, xs_4, xs_5, xs_6, xs_7]
    ws = [W_0, W_1, W_2, W_3, W_4, W_5, W_6, W_7]
    xf = [x.reshape(_B * _C, _L * _D) for x in xs]
    bmat = jnp.stack([b_0, b_1, b_2, b_3, b_4, b_5, b_6, b_7], axis=0)

    col0 = (_L - 1) * _D // _DBLK

    x_spec = pl.BlockSpec((_ROWS, _DBLK), lambda b, k: (b, col0 + k))
    w_spec = pl.BlockSpec((_PRED, _DBLK), lambda b, k: (0, k))
    g_spec = pl.BlockSpec((_BB, _PS), lambda b, k: (b, 0))
    bias_spec = pl.BlockSpec((_PS, _PRED), lambda b, k: (0, 0))

    out = pl.pallas_call(
        _head_kernel,
        grid=(_NB, _NK),
        in_specs=[g_spec, bias_spec] + [x_spec] * _PS + [w_spec] * _PS,
        out_specs=pl.BlockSpec((_BB, _PRED, _C), lambda b, k: (b, 0, 0)),
        out_shape=jax.ShapeDtypeStruct((_B, _PRED, _C), jnp.float32),
        scratch_shapes=[pltpu.VMEM((_ROWS, _PRED), jnp.float32)],
        compiler_params=pltpu.CompilerParams(
            dimension_semantics=("arbitrary", "arbitrary"),
        ),
    )(gates, bmat, *xf, *ws)
    return out


# speedup vs baseline: 1.0038x; 1.0038x over previous
import jax
import jax.numpy as jnp
from jax.experimental import pallas as pl
from jax.experimental.pallas import tpu as pltpu

_B, _C, _L, _D = 64, 32, 4, 2048
_PRED = 720
_PS = 8
_DBLK = 256
_NK = _D // _DBLK
_NB = 4
_BB = _B // _NB
_ROWS = _BB * _C


def _head_kernel(gates_ref, bmat_ref, *refs):
    xs = refs[:_PS]
    ws = refs[_PS:2 * _PS]
    out_ref = refs[2 * _PS]
    acc_ref = refs[2 * _PS + 1]
    k = pl.program_id(1)

    g = jnp.maximum(gates_ref[...], 0.0)
    acc = jnp.zeros((_ROWS, _PRED), jnp.float32)
    for i in range(_PS):
        x = xs[i][...].reshape(_BB, _C, _DBLK)
        gi = g[:, i].reshape(_BB, 1, 1)
        gx = (x * gi).reshape(_ROWS, _DBLK).astype(jnp.bfloat16)
        acc = acc + jax.lax.dot_general(
            gx, ws[i][...].astype(jnp.bfloat16),
            (((1,), (1,)), ((), ())),
            preferred_element_type=jnp.float32,
        )

    @pl.when(k == 0)
    def _():
        acc_ref[...] = acc

    @pl.when(k != 0)
    def _():
        acc_ref[...] = acc_ref[...] + acc

    @pl.when(k == _NK - 1)
    def _():
        bias = jax.lax.dot_general(
            g, bmat_ref[...],
            (((1,), (0,)), ((), ())),
            preferred_element_type=jnp.float32,
        )
        total = acc_ref[...].reshape(_BB, _C, _PRED) + bias[:, None, :] + 1e-9
        out_ref[...] = jnp.transpose(total, (0, 2, 1))


def kernel(xs_0, xs_1, xs_2, xs_3, xs_4, xs_5, xs_6, xs_7, gates,
           W_0, W_1, W_2, W_3, W_4, W_5, W_6, W_7,
           b_0, b_1, b_2, b_3, b_4, b_5, b_6, b_7):
    xs = [xs_0, xs_1, xs_2, xs_3, xs_4, xs_5, xs_6, xs_7]
    ws = [W_0, W_1, W_2, W_3, W_4, W_5, W_6, W_7]
    xf = [x.reshape(_B * _C, _L * _D) for x in xs]
    bmat = jnp.stack([b_0, b_1, b_2, b_3, b_4, b_5, b_6, b_7], axis=0)

    col0 = (_L - 1) * _D // _DBLK

    x_spec = pl.BlockSpec((_ROWS, _DBLK), lambda b, k: (b, col0 + k))
    w_spec = pl.BlockSpec((_PRED, _DBLK), lambda b, k: (0, k))
    g_spec = pl.BlockSpec((_BB, _PS), lambda b, k: (b, 0))
    bias_spec = pl.BlockSpec((_PS, _PRED), lambda b, k: (0, 0))

    out = pl.pallas_call(
        _head_kernel,
        grid=(_NB, _NK),
        in_specs=[g_spec, bias_spec] + [x_spec] * _PS + [w_spec] * _PS,
        out_specs=pl.BlockSpec((_BB, _PRED, _C), lambda b, k: (b, 0, 0)),
        out_shape=jax.ShapeDtypeStruct((_B, _PRED, _C), jnp.float32),
        scratch_shapes=[pltpu.VMEM((_ROWS, _PRED), jnp.float32)],
        compiler_params=pltpu.CompilerParams(
            dimension_semantics=("arbitrary", "arbitrary"),
        ),
    )(gates, bmat, *xf, *ws)
    return out


# speedup vs baseline: 1.2548x; 1.2500x over previous
import jax
import jax.numpy as jnp
from jax.experimental import pallas as pl
from jax.experimental.pallas import tpu as pltpu

_B, _C, _L, _D = 64, 32, 4, 2048
_PRED = 720
_PS = 8
_DBLK = 512
_NK = _D // _DBLK
_NB = 4
_BB = _B // _NB
_ROWS = _BB * _C


def _head_kernel(gates_ref, bmat_ref, *refs):
    xs = refs[:_PS]
    ws = refs[_PS:2 * _PS]
    out_ref = refs[2 * _PS]
    accs = refs[2 * _PS + 1:]
    k = pl.program_id(1)

    for i in range(_PS):
        x = xs[i][...].reshape(_ROWS, _DBLK)
        y = jax.lax.dot_general(
            x, ws[i][...],
            (((1,), (1,)), ((), ())),
            preferred_element_type=jnp.float32,
        )

        @pl.when(k == 0)
        def _():
            accs[i][...] = y

        @pl.when(k != 0)
        def _():
            accs[i][...] = accs[i][...] + y

    @pl.when(k == _NK - 1)
    def _():
        g = jnp.maximum(gates_ref[...], 0.0)
        bias = jax.lax.dot_general(
            g, bmat_ref[...],
            (((1,), (0,)), ((), ())),
            preferred_element_type=jnp.float32,
        )
        total = bias[:, None, :] + 1e-9
        for i in range(_PS):
            gi = g[:, i].reshape(_BB, 1, 1)
            total = total + accs[i][...].reshape(_BB, _C, _PRED) * gi
        out_ref[...] = jnp.transpose(total, (0, 2, 1))


def kernel(xs_0, xs_1, xs_2, xs_3, xs_4, xs_5, xs_6, xs_7, gates,
           W_0, W_1, W_2, W_3, W_4, W_5, W_6, W_7,
           b_0, b_1, b_2, b_3, b_4, b_5, b_6, b_7):
    xs = [xs_0, xs_1, xs_2, xs_3, xs_4, xs_5, xs_6, xs_7]
    ws = [W_0, W_1, W_2, W_3, W_4, W_5, W_6, W_7]
    xl = [x[:, :, _L - 1, :].astype(jnp.bfloat16) for x in xs]
    wb = [w.astype(jnp.bfloat16) for w in ws]
    bmat = jnp.stack([b_0, b_1, b_2, b_3, b_4, b_5, b_6, b_7], axis=0)

    x_spec = pl.BlockSpec((_BB, _C, _DBLK), lambda b, k: (b, 0, k))
    w_spec = pl.BlockSpec((_PRED, _DBLK), lambda b, k: (0, k))
    g_spec = pl.BlockSpec((_BB, _PS), lambda b, k: (b, 0))
    bias_spec = pl.BlockSpec((_PS, _PRED), lambda b, k: (0, 0))

    out = pl.pallas_call(
        _head_kernel,
        grid=(_NB, _NK),
        in_specs=[g_spec, bias_spec] + [x_spec] * _PS + [w_spec] * _PS,
        out_specs=pl.BlockSpec((_BB, _PRED, _C), lambda b, k: (b, 0, 0)),
        out_shape=jax.ShapeDtypeStruct((_B, _PRED, _C), jnp.float32),
        scratch_shapes=[pltpu.VMEM((_ROWS, _PRED), jnp.float32)
                        for _ in range(_PS)],
        compiler_params=pltpu.CompilerParams(
            dimension_semantics=("arbitrary", "arbitrary"),
        ),
    )(gates, bmat, *xl, *wb)
    return out


# speedup vs baseline: 4.5352x; 3.6144x over previous
import jax
import jax.numpy as jnp
from jax.experimental import pallas as pl
from jax.experimental.pallas import tpu as pltpu

_B, _C, _L, _D = 64, 32, 4, 2048
_PRED = 720
_PS = 8
_DBLK = 256
_NK = _D // _DBLK
_NB = 2
_BB = _B // _NB
_ROWS = _BB * _C
_NSTEPS = _NB * _NK


def _x_copy(xs, xbuf, sem, slot, b, k, i):
    return pltpu.make_async_copy(
        xs[i].at[pl.ds(b * _BB, _BB), :, _L - 1, pl.ds(k * _DBLK, _DBLK)],
        xbuf.at[slot, i],
        sem.at[slot, i],
    )


def _head_kernel(gates_ref, bmat_ref, *refs):
    xs = refs[:_PS]
    ws = refs[_PS:2 * _PS]
    out_ref = refs[2 * _PS]
    acc_ref, xbuf, sem = refs[2 * _PS + 1:]
    b = pl.program_id(0)
    k = pl.program_id(1)
    s = b * _NK + k
    slot = jax.lax.rem(s, 2)

    @pl.when(s == 0)
    def _():
        for i in range(_PS):
            _x_copy(xs, xbuf, sem, 0, b, k, i).start()

    @pl.when(s + 1 < _NSTEPS)
    def _():
        s1 = s + 1
        b1 = jax.lax.div(s1, _NK)
        k1 = jax.lax.rem(s1, _NK)
        slot1 = jax.lax.rem(s1, 2)
        for i in range(_PS):
            _x_copy(xs, xbuf, sem, slot1, b1, k1, i).start()

    for i in range(_PS):
        _x_copy(xs, xbuf, sem, slot, b, k, i).wait()

    g = jnp.maximum(gates_ref[...], 0.0)
    acc = jnp.zeros((_ROWS, _PRED), jnp.float32)
    for i in range(_PS):
        x = xbuf[slot, i]
        gi = g[:, i].reshape(_BB, 1, 1)
        gx = (x * gi).reshape(_ROWS, _DBLK).astype(jnp.bfloat16)
        acc = acc + jax.lax.dot_general(
            gx, ws[i][...].astype(jnp.bfloat16),
            (((1,), (1,)), ((), ())),
            preferred_element_type=jnp.float32,
        )

    @pl.when(k == 0)
    def _():
        acc_ref[...] = acc

    @pl.when(k != 0)
    def _():
        acc_ref[...] = acc_ref[...] + acc

    @pl.when(k == _NK - 1)
    def _():
        bias = jax.lax.dot_general(
            g, bmat_ref[...],
            (((1,), (0,)), ((), ())),
            preferred_element_type=jnp.float32,
        )
        total = acc_ref[...].reshape(_BB, _C, _PRED) + bias[:, None, :] + 1e-9
        out_ref[...] = jnp.transpose(total, (0, 2, 1))


def kernel(xs_0, xs_1, xs_2, xs_3, xs_4, xs_5, xs_6, xs_7, gates,
           W_0, W_1, W_2, W_3, W_4, W_5, W_6, W_7,
           b_0, b_1, b_2, b_3, b_4, b_5, b_6, b_7):
    xs = [xs_0, xs_1, xs_2, xs_3, xs_4, xs_5, xs_6, xs_7]
    ws = [W_0, W_1, W_2, W_3, W_4, W_5, W_6, W_7]
    bmat = jnp.stack([b_0, b_1, b_2, b_3, b_4, b_5, b_6, b_7], axis=0)

    x_spec = pl.BlockSpec(memory_space=pltpu.MemorySpace.HBM)
    w_spec = pl.BlockSpec((_PRED, _DBLK), lambda b, k: (0, k))
    g_spec = pl.BlockSpec((_BB, _PS), lambda b, k: (b, 0))
    bias_spec = pl.BlockSpec((_PS, _PRED), lambda b, k: (0, 0))

    out = pl.pallas_call(
        _head_kernel,
        grid=(_NB, _NK),
        in_specs=[g_spec, bias_spec] + [x_spec] * _PS + [w_spec] * _PS,
        out_specs=pl.BlockSpec((_BB, _PRED, _C), lambda b, k: (b, 0, 0)),
        out_shape=jax.ShapeDtypeStruct((_B, _PRED, _C), jnp.float32),
        scratch_shapes=[
            pltpu.VMEM((_ROWS, _PRED), jnp.float32),
            pltpu.VMEM((2, _PS, _BB, _C, _DBLK), jnp.float32),
            pltpu.SemaphoreType.DMA((2, _PS)),
        ],
        compiler_params=pltpu.CompilerParams(
            dimension_semantics=("arbitrary", "arbitrary"),
        ),
    )(gates, bmat, *xs, *ws)
    return out
